# baseline (speedup 1.0000x reference)
import functools
import math

import jax
import jax.numpy as jnp
from jax import lax
from jax.experimental import pallas as pl
from jax.experimental.pallas import tpu as pltpu
from jax.experimental.pallas import tpu_sc as plsc

B, O, N = 4, 256, 8192
TOKEN_DIM, D, TOP_K, H = 512, 256, 128, 8
DH = D // H
BN = 512
NB = N // BN
VSW = 384
ROWS = B * O
NC, NS = 2, 16
NW = NC * NS
RPW = ROWS // NW
HB1 = 2048
EPS = 1e-5


def _dotT(a, b):
    return lax.dot_general(a, b, (((1,), (1,)), ((), ())),
                           preferred_element_type=jnp.float32)


def _ln(x, g, b):
    m = x.mean(-1, keepdims=True)
    v = ((x - m) ** 2).mean(-1, keepdims=True)
    return (x - m) / jnp.sqrt(v + EPS) * g + b


def _t1_body(tok_ref, act_ref, wp_ref, bp_ref, tng_ref, tnb_ref, pq_ref,
             wq_ref, wk_ref, wv_ref, bq_ref, bk_ref, bv_ref,
             vs_ref, ua_ref, s_ref):
    n = pl.program_id(1)
    tok = tok_ref[0]
    tp = _ln(_dotT(tok, wp_ref[...]) + bp_ref[...], tng_ref[...], tnb_ref[...])

    v = _dotT(tp, wv_ref[...]) + bv_ref[...]
    vs_ref[:, pl.ds(0, D)] = v

    scale = 1.0 / math.sqrt(DH)
    q = pq_ref[0, 0]
    qv = _dotT(q[None, :], wq_ref[...]) + bq_ref[...][None, :]
    rowid = lax.broadcasted_iota(jnp.int32, (16, D), 0)
    colid = lax.broadcasted_iota(jnp.int32, (16, D), 1)
    esel = jnp.where((colid >> 5) == rowid, scale, 0.0)
    a16 = lax.dot_general(esel * qv, wk_ref[...], (((1,), (0,)), ((), ())),
                          preferred_element_type=jnp.float32)
    vs_ref[:, pl.ds(D, 16)] = _dotT(tp, a16)

    act = act_ref[0]
    partial = lax.dot_general(act, tp, (((1,), (0,)), ((), ())),
                              preferred_element_type=jnp.float32)
    rs = act.sum(axis=1)

    @pl.when(n == 0)
    def _():
        ua_ref[0] = partial
        s_ref[0, 0] = rs

    @pl.when(n != 0)
    def _():
        ua_ref[0] += partial
        s_ref[0, 0] += rs


def _t1(activation, tokens, wp, bp, tng, tnb, pq, wq, wk, wv, bq, bk, bv):
    return pl.pallas_call(
        _t1_body,
        grid=(B, NB),
        in_specs=[
            pl.BlockSpec((1, BN, TOKEN_DIM), lambda b, n: (b, n, 0)),
            pl.BlockSpec((1, O, BN), lambda b, n: (b, 0, n)),
            pl.BlockSpec((D, TOKEN_DIM), lambda b, n: (0, 0)),
            pl.BlockSpec((D,), lambda b, n: (0,)),
            pl.BlockSpec((D,), lambda b, n: (0,)),
            pl.BlockSpec((D,), lambda b, n: (0,)),
            pl.BlockSpec((1, 1, D), lambda b, n: (0, 0, 0)),
            pl.BlockSpec((D, D), lambda b, n: (0, 0)),
            pl.BlockSpec((D, D), lambda b, n: (0, 0)),
            pl.BlockSpec((D, D), lambda b, n: (0, 0)),
            pl.BlockSpec((D,), lambda b, n: (0,)),
            pl.BlockSpec((D,), lambda b, n: (0,)),
            pl.BlockSpec((D,), lambda b, n: (0,)),
        ],
        out_specs=[
            pl.BlockSpec((BN, VSW), lambda b, n: (b * NB + n, 0)),
            pl.BlockSpec((1, O, D), lambda b, n: (b, 0, 0)),
            pl.BlockSpec((1, 1, O), lambda b, n: (b, 0, 0)),
        ],
        out_shape=[
            jax.ShapeDtypeStruct((B * N, VSW), jnp.float32),
            jax.ShapeDtypeStruct((B, O, D), jnp.float32),
            jax.ShapeDtypeStruct((B, 1, O), jnp.float32),
        ],
    )(tokens, activation, wp, bp, tng, tnb, pq, wq, wk, wv, bq, bk, bv)


def _walk(hist_ref, nchunks, kneed):
    lanes = lax.iota(jnp.int32, 16)

    def cond(st):
        m, tot, found, dst, cgt = st
        return jnp.logical_and(m >= 0, found == 0)

    def body(st):
        m, tot, found, dst, cgt = st
        chunk = hist_ref[pl.ds(m * 16, 16)]
        csum = jnp.sum(chunk)

        def in_chunk(_):
            rev = lax.rev(chunk, (0,))
            cum = jnp.cumsum(rev)
            crossed = (tot + cum) >= kneed
            istar = jnp.max(plsc.all_reduce_ffs(crossed))
            above = jnp.sum(jnp.where(lanes < istar, rev, 0))
            return (m - 1, tot, jnp.int32(1), m * 16 + 15 - istar,
                    tot + above)

        def skip(_):
            return (m - 1, tot + csum, found, dst, cgt)

        return lax.cond(tot + csum >= kneed, in_chunk, skip, 0)

    st0 = (jnp.int32(nchunks - 1), jnp.int32(0), jnp.int32(0), jnp.int32(0),
           jnp.int32(0))
    _, _, _, dstar, cgt = lax.while_loop(cond, body, st0)
    return dstar, cgt


def _s1a_call(act2):
    mesh = plsc.VectorSubcoreMesh(core_axis_name="c", subcore_axis_name="s")

    @functools.partial(
        pl.kernel,
        mesh=mesh,
        compiler_params=pltpu.CompilerParams(needs_layout_passes=False),
        out_type=jax.ShapeDtypeStruct((ROWS, TOP_K), jnp.int32),
        scratch_types=[
            pltpu.VMEM((N,), jnp.float32),
            pltpu.VMEM((N,), jnp.float32),
            pltpu.VMEM((HB1,), jnp.int32),
            pltpu.VMEM((TOP_K + 16,), jnp.int32),
            pltpu.VMEM((N + 16,), jnp.int32),
            pltpu.VMEM((N + 16,), jnp.int32),
            pltpu.SemaphoreType.DMA,
            pltpu.SemaphoreType.DMA,
        ],
    )
    def s1a(act_hbm, idx_hbm, act_a, act_b, hist_v, idx_s, ci_a,
            ci_b, sem_a, sem_b):
        wid = lax.axis_index("s") * NC + lax.axis_index("c")
        base = wid * RPW
        lanes = lax.iota(jnp.int32, 16)
        ones = jnp.ones((16,), jnp.int32)

        def radix_row(act_v, row):
            b_off = (row >> 8) << 13

            def zb(i, _c):
                hist_v[pl.ds(i * 16, 16)] = jnp.zeros((16,), jnp.int32)
                return 0
            lax.fori_loop(0, HB1 // 16, zb, 0, unroll=8)

            def hb(i, _c):
                k = plsc.bitcast(act_v[pl.ds(i * 16, 16)], jnp.int32)
                plsc.addupdate_scatter(hist_v, [k >> 19], ones)
                return 0
            lax.fori_loop(0, N // 16, hb, 0, unroll=8)

            dstar, cgt = _walk(hist_v, HB1 // 16, TOP_K)
            kp = TOP_K - cgt

            zero16 = jnp.zeros((16,), jnp.int32)

            def em1(i, st):
                off_in, off_eq = st
                k = plsc.bitcast(act_v[pl.ds(i * 16, 16)], jnp.int32)
                d = k >> 19
                gidx = b_off + i * 16 + lanes
                m_in = d > dstar
                m_eq = d == dstar
                cs_in = jnp.cumsum(m_in.astype(jnp.int32))
                plsc.store_scatter(idx_s, [off_in + cs_in - 1], gidx,
                                   mask=m_in)
                cs_eq = jnp.cumsum(m_eq.astype(jnp.int32))
                plsc.store_scatter(ci_a, [off_eq + cs_eq - 1], gidx,
                                   mask=m_eq)
                return (off_in + plsc.all_reduce_population_count(m_in),
                        off_eq + plsc.all_reduce_population_count(m_eq))

            off_in_v, ccnt_v = lax.fori_loop(0, N // 16, em1,
                                             (zero16, zero16), unroll=4)
            ccnt = ccnt_v[0]

            for (shift, nb), (si, di) in zip(
                    [(11, 256), (3, 256), (0, 8)],
                    [(ci_a, ci_b), (ci_b, ci_a), (ci_a, ci_b)]):

                def zb2(i, _c):
                    hist_v[pl.ds(i * 16, 16)] = jnp.zeros((16,), jnp.int32)
                    return 0
                lax.fori_loop(0, nb // 16, zb2, 0)

                def hb2(i, _c, si=si, shift=shift, nb=nb, cc=ccnt,
                        bo=b_off):
                    gi = si[pl.ds(i * 16, 16)]
                    valid = (i * 16 + lanes) < cc
                    kf = plsc.load_gather(act_v, [gi - bo], mask=valid)
                    dd = (plsc.bitcast(kf, jnp.int32) >> shift) & (nb - 1)
                    plsc.addupdate_scatter(hist_v, [dd], ones, mask=valid)
                    return 0
                lax.fori_loop(0, (ccnt + 15) // 16, hb2, 0)

                dstar, cgt = _walk(hist_v, nb // 16, kp)
                kp = kp - cgt

                def em2(i, st, si=si, di=di, shift=shift, nb=nb, cc=ccnt,
                        ds=dstar, bo=b_off):
                    off_i, off_e = st
                    gi = si[pl.ds(i * 16, 16)]
                    valid = (i * 16 + lanes) < cc
                    kf = plsc.load_gather(act_v, [gi - bo], mask=valid)
                    dd = (plsc.bitcast(kf, jnp.int32) >> shift) & (nb - 1)
                    m_in = valid & (dd > ds)
                    m_eq = valid & (dd == ds)
                    cs_in = jnp.cumsum(m_in.astype(jnp.int32))
                    plsc.store_scatter(idx_s, [off_i + cs_in - 1], gi,
                                       mask=m_in)
                    cs_eq = jnp.cumsum(m_eq.astype(jnp.int32))
                    plsc.store_scatter(di, [off_e + cs_eq - 1], gi,
                                       mask=m_eq)
                    return (off_i + plsc.all_reduce_population_count(m_in),
                            off_e + plsc.all_reduce_population_count(m_eq))

                off_in_v, ccnt_v = lax.fori_loop(0, (ccnt + 15) // 16, em2,
                                                 (off_in_v, zero16))
                ccnt = ccnt_v[0]

            fi = ci_b

            def fe(i, _c):
                gi = fi[pl.ds(i * 16, 16)]
                valid = (i * 16 + lanes) < kp
                plsc.store_scatter(idx_s, [off_in_v + i * 16 + lanes], gi,
                                   mask=valid)
                return 0
            lax.fori_loop(0, (kp + 15) // 16, fe, 0)

            pltpu.sync_copy(idx_s.at[pl.ds(0, TOP_K)], idx_hbm.at[row])

        def issue_act(local, buf, sem):
            src_row = base + jnp.minimum(local, RPW - 1)
            pltpu.async_copy(act_hbm.at[src_row], buf, sem)

        def wait_act(buf, sem):
            pltpu.make_async_copy(act_hbm.at[base], buf, sem).wait()

        issue_act(jnp.int32(0), act_a, sem_a)

        def gbody(g, _):
            r0 = 2 * g
            issue_act(r0 + 1, act_b, sem_b)
            wait_act(act_a, sem_a)
            radix_row(act_a, base + r0)
            issue_act(r0 + 2, act_a, sem_a)
            wait_act(act_b, sem_b)
            radix_row(act_b, base + r0 + 1)
            return 0

        lax.fori_loop(0, RPW // 2, gbody, 0)
        wait_act(act_a, sem_a)

    return s1a(act2)


def _s1b_call(idxt, vs):
    mesh = plsc.VectorSubcoreMesh(core_axis_name="c", subcore_axis_name="s")

    @functools.partial(
        pl.kernel,
        mesh=mesh,
        compiler_params=pltpu.CompilerParams(needs_layout_passes=False),
        out_type=jax.ShapeDtypeStruct((ROWS, D), jnp.float32),
        scratch_types=[
            pltpu.VMEM((RPW, TOP_K), jnp.int32),
            pltpu.VMEM((TOP_K, VSW), jnp.float32),
            pltpu.VMEM((TOP_K, VSW), jnp.float32),
            pltpu.VMEM((D,), jnp.float32),
            pltpu.SemaphoreType.DMA,
            pltpu.SemaphoreType.DMA,
        ],
    )
    def s1b(idx_hbm, vs_hbm, out_hbm, idx_all, rows_a, rows_b, outr_v,
            sem_a, sem_b):
        wid = lax.axis_index("s") * NC + lax.axis_index("c")
        base = wid * RPW
        pltpu.sync_copy(idx_hbm.at[pl.ds(base, RPW)], idx_all)

        def issue_gather(local, buf, sem):
            src = idx_all.at[jnp.minimum(local, RPW - 1)]
            pltpu.async_copy(vs_hbm.at[src], buf, sem)

        def wait_gather(buf, sem):
            pltpu.make_async_copy(vs_hbm.at[idx_all.at[0]], buf, sem).wait()

        def pool_row(rows_v, row):
            def mb(j, m):
                return jnp.maximum(m, rows_v[j, pl.ds(D, 16)])
            mx = lax.fori_loop(1, TOP_K, mb, rows_v[0, pl.ds(D, 16)],
                               unroll=8)

            def eb(j, den):
                e = jnp.exp(rows_v[j, pl.ds(D, 16)] - mx)
                rows_v[j, pl.ds(D, 16)] = e
                return den + e
            den = lax.fori_loop(0, TOP_K, eb, jnp.zeros((16,), jnp.float32),
                                unroll=8)

            def wb(j, accs):
                ev = rows_v[j, pl.ds(D, 16)]
                es = [ev[h] for h in range(H)]
                return tuple(
                    accs[c] + es[c >> 1] * rows_v[j, pl.ds(c * 16, 16)]
                    for c in range(16))
            acc0 = tuple(jnp.zeros((16,), jnp.float32) for _ in range(16))
            accs = lax.fori_loop(0, TOP_K, wb, acc0)
            for c in range(16):
                outr_v[pl.ds(c * 16, 16)] = accs[c] / den[c >> 1]
            pltpu.sync_copy(outr_v, out_hbm.at[row])

        issue_gather(jnp.int32(0), rows_a, sem_a)

        def gbody(g, _):
            r0 = 2 * g
            issue_gather(r0 + 1, rows_b, sem_b)
            wait_gather(rows_a, sem_a)
            pool_row(rows_a, base + r0)
            issue_gather(r0 + 2, rows_a, sem_a)
            wait_gather(rows_b, sem_b)
            pool_row(rows_b, base + r0 + 1)
            return 0

        lax.fori_loop(0, RPW // 2, gbody, 0)
        wait_gather(rows_a, sem_a)

    return s1b(idxt, vs)


def _t2_body(ua_ref, s_ref, at_ref, ow_ref, ob_ref, png_ref, pnb_ref,
             ong_ref, onb_ref, out_ref):
    s = jnp.maximum(s_ref[0, 0], 1e-8)
    wa = ua_ref[0] / s[:, None]
    pooled = _dotT(at_ref[0], ow_ref[...]) + ob_ref[...]
    pooled = _ln(pooled, png_ref[...], pnb_ref[...])
    out_ref[0] = _ln(wa + pooled, ong_ref[...], onb_ref[...])


def _t2(ua, s, attn, ow, ob, png, pnb, ong, onb):
    return pl.pallas_call(
        _t2_body,
        grid=(B,),
        in_specs=[
            pl.BlockSpec((1, O, D), lambda b: (b, 0, 0)),
            pl.BlockSpec((1, 1, O), lambda b: (b, 0, 0)),
            pl.BlockSpec((1, O, D), lambda b: (b, 0, 0)),
            pl.BlockSpec((D, D), lambda b: (0, 0)),
            pl.BlockSpec((D,), lambda b: (0,)),
            pl.BlockSpec((D,), lambda b: (0,)),
            pl.BlockSpec((D,), lambda b: (0,)),
            pl.BlockSpec((D,), lambda b: (0,)),
            pl.BlockSpec((D,), lambda b: (0,)),
        ],
        out_specs=pl.BlockSpec((1, O, D), lambda b: (b, 0, 0)),
        out_shape=jax.ShapeDtypeStruct((B, O, D), jnp.float32),
    )(ua, s, attn, ow, ob, png, pnb, ong, onb)


def kernel(activation, tokens, token_proj_w, token_proj_b, token_norm_g,
           token_norm_b, pool_query, in_proj_w, in_proj_b, out_proj_w,
           out_proj_b, pool_norm_g, pool_norm_b, out_norm_g, out_norm_b):
    wq, wk, wv = in_proj_w[:D], in_proj_w[D:2 * D], in_proj_w[2 * D:]
    bq, bk, bv = in_proj_b[:D], in_proj_b[D:2 * D], in_proj_b[2 * D:]

    vs, ua, s = _t1(activation, tokens, token_proj_w, token_proj_b,
                    token_norm_g, token_norm_b, pool_query,
                    wq, wk, wv, bq, bk, bv)
    idxt = _s1a_call(activation.reshape(ROWS, N))
    attn = _s1b_call(idxt, vs)
    return _t2(ua, s, attn.reshape(B, O, D), out_proj_w, out_proj_b,
               pool_norm_g, pool_norm_b, out_norm_g, out_norm_b)


# speedup vs baseline: 2.2238x; 2.2238x over previous
import functools
import math

import jax
import jax.numpy as jnp
from jax import lax
from jax.experimental import pallas as pl
from jax.experimental.pallas import tpu as pltpu
from jax.experimental.pallas import tpu_sc as plsc

B, O, N = 4, 256, 8192
TOKEN_DIM, D, TOP_K, H = 512, 256, 128, 8
DH = D // H
BN = 512
NB = N // BN
VSW = 384
ROWS = B * O
NC, NS = 2, 16
NW = NC * NS
RPW = ROWS // NW
HB1 = 2048
EPS = 1e-5


def _dotT(a, b):
    return lax.dot_general(a, b, (((1,), (1,)), ((), ())),
                           preferred_element_type=jnp.float32)


def _ln(x, g, b):
    m = x.mean(-1, keepdims=True)
    v = ((x - m) ** 2).mean(-1, keepdims=True)
    return (x - m) / jnp.sqrt(v + EPS) * g + b


def _t1_body(tok_ref, act_ref, wp_ref, bp_ref, tng_ref, tnb_ref, pq_ref,
             wq_ref, wk_ref, wv_ref, bq_ref, bk_ref, bv_ref,
             vs_ref, ua_ref, s_ref):
    n = pl.program_id(1)
    tok = tok_ref[0]
    tp = _ln(_dotT(tok, wp_ref[...]) + bp_ref[...], tng_ref[...], tnb_ref[...])

    v = _dotT(tp, wv_ref[...]) + bv_ref[...]
    vs_ref[:, pl.ds(0, D)] = v

    scale = 1.0 / math.sqrt(DH)
    q = pq_ref[0, 0]
    qv = _dotT(q[None, :], wq_ref[...]) + bq_ref[...][None, :]
    rowid = lax.broadcasted_iota(jnp.int32, (16, D), 0)
    colid = lax.broadcasted_iota(jnp.int32, (16, D), 1)
    esel = jnp.where((colid >> 5) == rowid, scale, 0.0)
    a16 = lax.dot_general(esel * qv, wk_ref[...], (((1,), (0,)), ((), ())),
                          preferred_element_type=jnp.float32)
    vs_ref[:, pl.ds(D, 16)] = _dotT(tp, a16)

    act = act_ref[0]
    partial = lax.dot_general(act, tp, (((1,), (0,)), ((), ())),
                              preferred_element_type=jnp.float32)
    rs = act.sum(axis=1)

    @pl.when(n == 0)
    def _():
        ua_ref[0] = partial
        s_ref[0, 0] = rs

    @pl.when(n != 0)
    def _():
        ua_ref[0] += partial
        s_ref[0, 0] += rs


def _t1(activation, tokens, wp, bp, tng, tnb, pq, wq, wk, wv, bq, bk, bv):
    return pl.pallas_call(
        _t1_body,
        grid=(B, NB),
        in_specs=[
            pl.BlockSpec((1, BN, TOKEN_DIM), lambda b, n: (b, n, 0)),
            pl.BlockSpec((1, O, BN), lambda b, n: (b, 0, n)),
            pl.BlockSpec((D, TOKEN_DIM), lambda b, n: (0, 0)),
            pl.BlockSpec((D,), lambda b, n: (0,)),
            pl.BlockSpec((D,), lambda b, n: (0,)),
            pl.BlockSpec((D,), lambda b, n: (0,)),
            pl.BlockSpec((1, 1, D), lambda b, n: (0, 0, 0)),
            pl.BlockSpec((D, D), lambda b, n: (0, 0)),
            pl.BlockSpec((D, D), lambda b, n: (0, 0)),
            pl.BlockSpec((D, D), lambda b, n: (0, 0)),
            pl.BlockSpec((D,), lambda b, n: (0,)),
            pl.BlockSpec((D,), lambda b, n: (0,)),
            pl.BlockSpec((D,), lambda b, n: (0,)),
        ],
        out_specs=[
            pl.BlockSpec((BN, VSW), lambda b, n: (b * NB + n, 0)),
            pl.BlockSpec((1, O, D), lambda b, n: (b, 0, 0)),
            pl.BlockSpec((1, 1, O), lambda b, n: (b, 0, 0)),
        ],
        out_shape=[
            jax.ShapeDtypeStruct((B * N, VSW), jnp.float32),
            jax.ShapeDtypeStruct((B, O, D), jnp.float32),
            jax.ShapeDtypeStruct((B, 1, O), jnp.float32),
        ],
    )(tokens, activation, wp, bp, tng, tnb, pq, wq, wk, wv, bq, bk, bv)


def _walk(hist_ref, nchunks, kneed):
    lanes = lax.iota(jnp.int32, 16)

    def cond(st):
        m, tot, found, dst, cgt = st
        return jnp.logical_and(m >= 0, found == 0)

    def body(st):
        m, tot, found, dst, cgt = st
        chunk = hist_ref[pl.ds(m * 16, 16)]
        csum = jnp.sum(chunk)

        def in_chunk(_):
            rev = lax.rev(chunk, (0,))
            cum = jnp.cumsum(rev)
            crossed = (tot + cum) >= kneed
            istar = jnp.max(plsc.all_reduce_ffs(crossed))
            above = jnp.sum(jnp.where(lanes < istar, rev, 0))
            return (m - 1, tot, jnp.int32(1), m * 16 + 15 - istar,
                    tot + above)

        def skip(_):
            return (m - 1, tot + csum, found, dst, cgt)

        return lax.cond(tot + csum >= kneed, in_chunk, skip, 0)

    st0 = (jnp.int32(nchunks - 1), jnp.int32(0), jnp.int32(0), jnp.int32(0),
           jnp.int32(0))
    _, _, _, dstar, cgt = lax.while_loop(cond, body, st0)
    return dstar, cgt


def _s1a_call(act2):
    mesh = plsc.VectorSubcoreMesh(core_axis_name="c", subcore_axis_name="s")

    @functools.partial(
        pl.kernel,
        mesh=mesh,
        compiler_params=pltpu.CompilerParams(needs_layout_passes=False),
        out_type=jax.ShapeDtypeStruct((ROWS, TOP_K), jnp.int32),
        scratch_types=[
            pltpu.VMEM((N,), jnp.float32),
            pltpu.VMEM((N,), jnp.float32),
            pltpu.VMEM((HB1,), jnp.int32),
            pltpu.VMEM((TOP_K + 16,), jnp.int32),
            pltpu.VMEM((N + 16,), jnp.int32),
            pltpu.VMEM((N + 16,), jnp.int32),
            pltpu.SemaphoreType.DMA,
            pltpu.SemaphoreType.DMA,
        ],
    )
    def s1a(act_hbm, idx_hbm, act_a, act_b, hist_v, idx_s, ci_a,
            ci_b, sem_a, sem_b):
        wid = lax.axis_index("s") * NC + lax.axis_index("c")
        base = wid * RPW
        lanes = lax.iota(jnp.int32, 16)
        ones = jnp.ones((16,), jnp.int32)

        def radix_row(act_v, row):
            b_off = (row >> 8) << 13

            @plsc.parallel_loop(0, HB1 // 16, unroll=8)
            def _zb(i):
                hist_v[pl.ds(i * 16, 16)] = jnp.zeros((16,), jnp.int32)

            @plsc.parallel_loop(0, N // 16, unroll=8)
            def _hb(i):
                k = plsc.bitcast(act_v[pl.ds(i * 16, 16)], jnp.int32)
                plsc.addupdate_scatter(hist_v, [k >> 19], ones)

            ds1, cgt1 = _walk(hist_v, HB1 // 16, TOP_K)
            kp = TOP_K - cgt1

            zero16 = jnp.zeros((16,), jnp.int32)

            def em1(i, off):
                k = plsc.bitcast(act_v[pl.ds(i * 16, 16)], jnp.int32)
                m = (k >> 19) >= ds1
                cs = jnp.cumsum(m.astype(jnp.int32))
                plsc.store_scatter(ci_a, [off + cs - 1],
                                   b_off + i * 16 + lanes, mask=m)
                return off + plsc.all_reduce_population_count(m)

            ccnt_v = plsc.parallel_loop(0, N // 16, unroll=4,
                                        carry=zero16)(em1)
            ccnt = ccnt_v[0]

            off_in_v = zero16
            for pi, ((shift, nb), (si, di)) in enumerate(zip(
                    [(11, 256), (3, 256), (0, 8)],
                    [(ci_a, ci_b), (ci_b, ci_a), (ci_a, ci_b)])):

                def zb2(i, _c):
                    hist_v[pl.ds(i * 16, 16)] = jnp.zeros((16,), jnp.int32)
                    return 0
                lax.fori_loop(0, nb // 16, zb2, 0)

                def hb2(i, _c, si=si, shift=shift, nb=nb, cc=ccnt,
                        bo=b_off, pi=pi):
                    gi = si[pl.ds(i * 16, 16)]
                    valid = (i * 16 + lanes) < cc
                    kf = plsc.load_gather(act_v, [gi - bo], mask=valid)
                    kk = plsc.bitcast(kf, jnp.int32)
                    if pi == 0:
                        valid = valid & ((kk >> 19) == ds1)
                    dd = (kk >> shift) & (nb - 1)
                    plsc.addupdate_scatter(hist_v, [dd], ones, mask=valid)
                    return 0
                lax.fori_loop(0, (ccnt + 15) // 16, hb2, 0)

                ds2, cgt = _walk(hist_v, nb // 16, kp)
                kp = kp - cgt

                def em2(i, st, si=si, di=di, shift=shift, nb=nb, cc=ccnt,
                        ds=ds2, bo=b_off, pi=pi):
                    off_i, off_e = st
                    gi = si[pl.ds(i * 16, 16)]
                    valid = (i * 16 + lanes) < cc
                    kf = plsc.load_gather(act_v, [gi - bo], mask=valid)
                    kk = plsc.bitcast(kf, jnp.int32)
                    dd = (kk >> shift) & (nb - 1)
                    if pi == 0:
                        d1 = kk >> 19
                        m_in = valid & ((d1 > ds1) | ((d1 == ds1) & (dd > ds)))
                        m_eq = valid & (d1 == ds1) & (dd == ds)
                    else:
                        m_in = valid & (dd > ds)
                        m_eq = valid & (dd == ds)
                    cs_in = jnp.cumsum(m_in.astype(jnp.int32))
                    plsc.store_scatter(idx_s, [off_i + cs_in - 1], gi,
                                       mask=m_in)
                    cs_eq = jnp.cumsum(m_eq.astype(jnp.int32))
                    plsc.store_scatter(di, [off_e + cs_eq - 1], gi,
                                       mask=m_eq)
                    return (off_i + plsc.all_reduce_population_count(m_in),
                            off_e + plsc.all_reduce_population_count(m_eq))

                off_in_v, ccnt_v = lax.fori_loop(0, (ccnt + 15) // 16, em2,
                                                 (off_in_v, zero16))
                ccnt = ccnt_v[0]

            fi = ci_b

            def fe(i, _c):
                gi = fi[pl.ds(i * 16, 16)]
                valid = (i * 16 + lanes) < kp
                plsc.store_scatter(idx_s, [off_in_v + i * 16 + lanes], gi,
                                   mask=valid)
                return 0
            lax.fori_loop(0, (kp + 15) // 16, fe, 0)

            pltpu.sync_copy(idx_s.at[pl.ds(0, TOP_K)], idx_hbm.at[row])

        def issue_act(local, buf, sem):
            src_row = base + jnp.minimum(local, RPW - 1)
            pltpu.async_copy(act_hbm.at[src_row], buf, sem)

        def wait_act(buf, sem):
            pltpu.make_async_copy(act_hbm.at[base], buf, sem).wait()

        issue_act(jnp.int32(0), act_a, sem_a)

        def gbody(g, _):
            r0 = 2 * g
            issue_act(r0 + 1, act_b, sem_b)
            wait_act(act_a, sem_a)
            radix_row(act_a, base + r0)
            issue_act(r0 + 2, act_a, sem_a)
            wait_act(act_b, sem_b)
            radix_row(act_b, base + r0 + 1)
            return 0

        lax.fori_loop(0, RPW // 2, gbody, 0)
        wait_act(act_a, sem_a)

    return s1a(act2)


def _s1b_call(idxt, vs):
    mesh = plsc.VectorSubcoreMesh(core_axis_name="c", subcore_axis_name="s")

    @functools.partial(
        pl.kernel,
        mesh=mesh,
        compiler_params=pltpu.CompilerParams(needs_layout_passes=False),
        out_type=jax.ShapeDtypeStruct((ROWS, D), jnp.float32),
        scratch_types=[
            pltpu.VMEM((RPW, TOP_K), jnp.int32),
            pltpu.VMEM((TOP_K, VSW), jnp.float32),
            pltpu.VMEM((TOP_K, VSW), jnp.float32),
            pltpu.VMEM((D,), jnp.float32),
            pltpu.SemaphoreType.DMA,
            pltpu.SemaphoreType.DMA,
        ],
    )
    def s1b(idx_hbm, vs_hbm, out_hbm, idx_all, rows_a, rows_b, outr_v,
            sem_a, sem_b):
        wid = lax.axis_index("s") * NC + lax.axis_index("c")
        base = wid * RPW
        pltpu.sync_copy(idx_hbm.at[pl.ds(base, RPW)], idx_all)

        def issue_gather(local, buf, sem):
            src = idx_all.at[jnp.minimum(local, RPW - 1)]
            pltpu.async_copy(vs_hbm.at[src], buf, sem)

        def wait_gather(buf, sem):
            pltpu.make_async_copy(vs_hbm.at[idx_all.at[0]], buf, sem).wait()

        def pool_row(rows_v, row):
            def mb(j, m):
                return jnp.maximum(m, rows_v[j, pl.ds(D, 16)])
            mx = lax.fori_loop(1, TOP_K, mb, rows_v[0, pl.ds(D, 16)],
                               unroll=8)

            def eb(j, den):
                e = jnp.exp(rows_v[j, pl.ds(D, 16)] - mx)
                rows_v[j, pl.ds(D, 16)] = e
                return den + e
            den = lax.fori_loop(0, TOP_K, eb, jnp.zeros((16,), jnp.float32),
                                unroll=8)

            def wb(j, accs):
                ev = rows_v[j, pl.ds(D, 16)]
                es = [ev[h] for h in range(H)]
                return tuple(
                    accs[c] + es[c >> 1] * rows_v[j, pl.ds(c * 16, 16)]
                    for c in range(16))
            acc0 = tuple(jnp.zeros((16,), jnp.float32) for _ in range(16))
            accs = lax.fori_loop(0, TOP_K, wb, acc0)
            for c in range(16):
                outr_v[pl.ds(c * 16, 16)] = accs[c] / den[c >> 1]
            pltpu.sync_copy(outr_v, out_hbm.at[row])

        issue_gather(jnp.int32(0), rows_a, sem_a)

        def gbody(g, _):
            r0 = 2 * g
            issue_gather(r0 + 1, rows_b, sem_b)
            wait_gather(rows_a, sem_a)
            pool_row(rows_a, base + r0)
            issue_gather(r0 + 2, rows_a, sem_a)
            wait_gather(rows_b, sem_b)
            pool_row(rows_b, base + r0 + 1)
            return 0

        lax.fori_loop(0, RPW // 2, gbody, 0)
        wait_gather(rows_a, sem_a)

    return s1b(idxt, vs)


def _t2_body(ua_ref, s_ref, at_ref, ow_ref, ob_ref, png_ref, pnb_ref,
             ong_ref, onb_ref, out_ref):
    s = jnp.maximum(s_ref[0, 0], 1e-8)
    wa = ua_ref[0] / s[:, None]
    pooled = _dotT(at_ref[0], ow_ref[...]) + ob_ref[...]
    pooled = _ln(pooled, png_ref[...], pnb_ref[...])
    out_ref[0] = _ln(wa + pooled, ong_ref[...], onb_ref[...])


def _t2(ua, s, attn, ow, ob, png, pnb, ong, onb):
    return pl.pallas_call(
        _t2_body,
        grid=(B,),
        in_specs=[
            pl.BlockSpec((1, O, D), lambda b: (b, 0, 0)),
            pl.BlockSpec((1, 1, O), lambda b: (b, 0, 0)),
            pl.BlockSpec((1, O, D), lambda b: (b, 0, 0)),
            pl.BlockSpec((D, D), lambda b: (0, 0)),
            pl.BlockSpec((D,), lambda b: (0,)),
            pl.BlockSpec((D,), lambda b: (0,)),
            pl.BlockSpec((D,), lambda b: (0,)),
            pl.BlockSpec((D,), lambda b: (0,)),
            pl.BlockSpec((D,), lambda b: (0,)),
        ],
        out_specs=pl.BlockSpec((1, O, D), lambda b: (b, 0, 0)),
        out_shape=jax.ShapeDtypeStruct((B, O, D), jnp.float32),
    )(ua, s, attn, ow, ob, png, pnb, ong, onb)


def kernel(activation, tokens, token_proj_w, token_proj_b, token_norm_g,
           token_norm_b, pool_query, in_proj_w, in_proj_b, out_proj_w,
           out_proj_b, pool_norm_g, pool_norm_b, out_norm_g, out_norm_b):
    wq, wk, wv = in_proj_w[:D], in_proj_w[D:2 * D], in_proj_w[2 * D:]
    bq, bk, bv = in_proj_b[:D], in_proj_b[D:2 * D], in_proj_b[2 * D:]

    vs, ua, s = _t1(activation, tokens, token_proj_w, token_proj_b,
                    token_norm_g, token_norm_b, pool_query,
                    wq, wk, wv, bq, bk, bv)
    idxt = _s1a_call(activation.reshape(ROWS, N))
    attn = _s1b_call(idxt, vs)
    return _t2(ua, s, attn.reshape(B, O, D), out_proj_w, out_proj_b,
               pool_norm_g, pool_norm_b, out_norm_g, out_norm_b)
